# Initial kernel scaffold; baseline (speedup 1.0000x reference)
import jax
import jax.numpy as jnp
from jax.experimental import pallas as pl


def kernel(item_cnt, item):
    raise NotImplementedError("write your pallas kernel here")


# speedup vs baseline: 2.0452x; 2.0452x over previous
import functools

import jax
import jax.numpy as jnp
from jax import lax
from jax.experimental import pallas as pl
from jax.experimental.pallas import tpu as pltpu
from jax.experimental.pallas import tpu_sc as plsc

N_ROWS = 1000000
B_SIZE = 16384

NC = 2
NS = 16
NW = NC * NS
BPW = B_SIZE // NW
IDX_ROWS = BPW // 128

SLICE = 31744
UNROLL = 8
VREGS = SLICE // 16
N_ITER = VREGS // UNROLL


@functools.cache
def _sc_gather_and_max():
    mesh = plsc.VectorSubcoreMesh(
        core_axis_name="c", subcore_axis_name="s", num_cores=NC, num_subcores=NS
    )

    @functools.partial(
        pl.kernel,
        out_type=(
            jax.ShapeDtypeStruct((B_SIZE // 128, 128), jnp.float32),
            jax.ShapeDtypeStruct((NW, 16), jnp.float32),
        ),
        mesh=mesh,
        scratch_types=(
            pltpu.VMEM((IDX_ROWS, 128), jnp.int32),
            pltpu.VMEM((IDX_ROWS, 128), jnp.float32),
            pltpu.VMEM((SLICE,), jnp.float32),
            pltpu.VMEM((16,), jnp.float32),
            pltpu.SemaphoreType.DMA,
        ),
    )
    def k(item_hbm, tbl_hbm, outg_hbm, outp_hbm, idx_v, rows_v, tbl_v, pm_v, sem):
        wid = lax.axis_index("s") * NC + lax.axis_index("c")

        pltpu.sync_copy(item_hbm.at[pl.ds(wid * IDX_ROWS, IDX_ROWS)], idx_v)
        gathers = [
            pltpu.async_copy(tbl_hbm.at[idx_v.at[j]], rows_v.at[j], sem)
            for j in range(IDX_ROWS)
        ]

        start = jnp.minimum(wid * SLICE, N_ROWS - SLICE)
        pltpu.sync_copy(tbl_hbm.at[pl.ds(start, SLICE)], tbl_v)

        def body(i, accs):
            base = i * (UNROLL * 16)
            return tuple(
                jnp.maximum(accs[j], tbl_v[pl.ds(base + j * 16, 16)])
                for j in range(UNROLL)
            )

        init = tuple(
            jnp.full((16,), -jnp.inf, jnp.float32) for _ in range(UNROLL)
        )
        accs = lax.fori_loop(0, N_ITER, body, init)
        acc = functools.reduce(jnp.maximum, accs)

        pm_v[...] = acc
        pltpu.sync_copy(pm_v, outp_hbm.at[wid])

        for c in gathers:
            c.wait()
        pltpu.sync_copy(rows_v, outg_hbm.at[pl.ds(wid * IDX_ROWS, IDX_ROWS)])

    return k


def _combine_body(g_ref, p_ref, o_ref):
    g = g_ref[...]
    mc = jnp.maximum(jnp.max(p_ref[...]), jnp.max(g) + 1.0)
    o_ref[...] = (g + 1.0) / mc


def _tc_combine(gathered2d, partials):
    return pl.pallas_call(
        _combine_body,
        out_shape=jax.ShapeDtypeStruct((B_SIZE // 128, 128), jnp.float32),
    )(gathered2d, partials)


def kernel(item_cnt, item):
    tbl = item_cnt.reshape(N_ROWS)
    item2d = item.reshape(B_SIZE // 128, 128)
    gathered2d, partials = _sc_gather_and_max()(item2d, tbl)
    out2d = _tc_combine(gathered2d, partials)
    return out2d.reshape(B_SIZE)
